# Initial kernel scaffold; baseline (speedup 1.0000x reference)
import jax
import jax.numpy as jnp
from jax.experimental import pallas as pl


def kernel(x, w1, b1, w2, b2):
    raise NotImplementedError("write your pallas kernel here")


# speedup vs baseline: 1.2495x; 1.2495x over previous
import functools

import jax
import jax.numpy as jnp
from jax.experimental import pallas as pl
from jax.experimental.pallas import tpu as pltpu


def _fused_se_kernel(x_ref, w1t_ref, b1r_ref, w2t_ref, b2r_ref, o_ref, *,
                     inv_hw):
    hwt = x_ref.shape[-1]
    if hwt % 128 == 0 and hwt > 128:
        part = x_ref[:, :, 0:128].astype(jnp.float32)
        for g in range(1, hwt // 128):
            part = part + x_ref[:, :, g * 128:(g + 1) * 128].astype(jnp.float32)
        pooled = jnp.sum(part, axis=-1) * inv_hw
    else:
        pooled = jnp.sum(x_ref[...].astype(jnp.float32), axis=-1) * inv_hw

    y1 = jnp.dot(pooled, w1t_ref[...],
                 preferred_element_type=jnp.float32) + b1r_ref[...]
    y1 = jnp.maximum(y1, 0.0)
    y2 = jnp.dot(y1, w2t_ref[...],
                 preferred_element_type=jnp.float32) + b2r_ref[...]
    gate = jax.nn.sigmoid(y2).astype(o_ref.dtype)

    o_ref[...] = x_ref[...] * gate[..., None]


@jax.jit
def _ca_fused(x, w1, b1, w2, b2):
    B, C, H, W = x.shape
    HW = H * W
    mid = w1.shape[0]
    x_flat = x.reshape(B, C, HW)

    bt = 1
    nb = B // bt

    w1t = jnp.transpose(w1)
    w2t = jnp.transpose(w2)
    b1r = b1.reshape(1, mid)
    b2r = b2.reshape(1, C)
    inv_hw = 1.0 / float(HW)

    out = pl.pallas_call(
        functools.partial(_fused_se_kernel, inv_hw=inv_hw),
        out_shape=jax.ShapeDtypeStruct((B, C, HW), x.dtype),
        grid=(nb,),
        in_specs=[
            pl.BlockSpec((bt, C, HW), lambda b: (b, 0, 0)),
            pl.BlockSpec((C, mid), lambda b: (0, 0)),
            pl.BlockSpec((1, mid), lambda b: (0, 0)),
            pl.BlockSpec((mid, C), lambda b: (0, 0)),
            pl.BlockSpec((1, C), lambda b: (0, 0)),
        ],
        out_specs=pl.BlockSpec((bt, C, HW), lambda b: (b, 0, 0)),
        compiler_params=pltpu.CompilerParams(
            dimension_semantics=("parallel",)),
    )(x_flat, w1t, b1r, w2t, b2r)

    return out.reshape(B, C, H, W)


def kernel(x, w1, b1, w2, b2):
    return _ca_fused(x, w1, b1, w2, b2)
